# baseline (speedup 1.0000x reference)
import functools

import jax
import jax.numpy as jnp
from jax import lax
from jax.experimental import pallas as pl
from jax.experimental.pallas import tpu as pltpu
from jax.experimental.pallas import tpu_sc as plsc

N_NODES = 100000
NUM_TYPES = 128

NC = 2
NS = 16
NW = NC * NS

RPW = 3200
LAST_BASE = (NW - 1) * RPW
LAST_ROWS = N_NODES - LAST_BASE
C = 160
GROUPS = C // 16

_mesh = plsc.VectorSubcoreMesh(core_axis_name="c", subcore_axis_name="s")


@functools.partial(
    pl.kernel,
    mesh=_mesh,
    compiler_params=pltpu.CompilerParams(
        needs_layout_passes=False,
        disable_bounds_checks=True,
        disable_semaphore_checks=True,
    ),
    out_type=jax.ShapeDtypeStruct((N_NODES, NUM_TYPES), jnp.float32),
    scratch_types=[
        pltpu.VMEM((RPW,), jnp.int32),
        pltpu.VMEM((C, NUM_TYPES), jnp.float32),
        pltpu.VMEM((C, NUM_TYPES), jnp.float32),
        pltpu.SemaphoreType.DMA,
        pltpu.SemaphoreType.DMA,
    ],
)
def _onehot_sc(a_hbm, out_hbm, idx_v, buf0, buf1, sem0, sem1):
    wid = lax.axis_index("s") * NC + lax.axis_index("c")
    base = wid * RPW
    rows_w = jnp.minimum(RPW, N_NODES - base)
    npairs = rows_w // (2 * C)

    @pl.when(wid < NW - 1)
    def _():
        pltpu.make_async_copy(a_hbm.at[pl.ds(base, RPW)], idx_v, sem0).start()

    @pl.when(wid == NW - 1)
    def _():
        pltpu.make_async_copy(a_hbm.at[pl.ds(LAST_BASE, LAST_ROWS)],
                              idx_v.at[pl.ds(0, LAST_ROWS)], sem0).start()

    iota16 = lax.iota(jnp.int32, 16)
    ones16 = jnp.ones((16,), jnp.float32)
    zeros16 = jnp.zeros((16,), jnp.float32)
    bufs = (buf0, buf1)
    sems = (sem0, sem1)

    def zrow(r, carry):
        for k in range(NUM_TYPES // 16):
            buf0[r, pl.ds(k * 16, 16)] = zeros16
            buf1[r, pl.ds(k * 16, 16)] = zeros16
        return carry

    lax.fori_loop(0, C, zrow, 0)

    @pl.when(wid < NW - 1)
    def _():
        pltpu.make_async_copy(a_hbm.at[pl.ds(base, RPW)], idx_v, sem0).wait()

    @pl.when(wid == NW - 1)
    def _():
        pltpu.make_async_copy(a_hbm.at[pl.ds(LAST_BASE, LAST_ROWS)],
                              idx_v.at[pl.ds(0, LAST_ROWS)], sem0).wait()

    def scat(buf, ci, vals):
        def g_body(g, carry):
            a = idx_v[pl.ds(ci * C + g * 16, 16)]
            rows = g * 16 + iota16
            plsc.store_scatter(buf, [rows, a], vals)
            return carry

        lax.fori_loop(0, GROUPS, g_body, 0)

    def pair(p, carry):
        for b in range(2):
            ci = 2 * p + b
            buf = bufs[b]
            sem = sems[b]
            row0 = base + ci * C

            @pl.when(p > 0)
            def _():
                pltpu.make_async_copy(buf, out_hbm.at[pl.ds(row0, C)], sem).wait()
                scat(buf, ci - 2, zeros16)

            scat(buf, ci, ones16)
            pltpu.make_async_copy(buf, out_hbm.at[pl.ds(row0, C)], sem).start()
        return carry

    lax.fori_loop(0, npairs, pair, 0)

    @pl.when(rows_w - npairs * 2 * C >= C)
    def _():
        ci = npairs * 2
        row0 = base + ci * C
        pltpu.make_async_copy(buf0, out_hbm.at[pl.ds(row0, C)], sem0).wait()
        scat(buf0, ci - 2, zeros16)
        scat(buf0, ci, ones16)
        pltpu.make_async_copy(buf0, out_hbm.at[pl.ds(row0, C)], sem0).start()

    pltpu.make_async_copy(buf0, out_hbm.at[pl.ds(0, C)], sem0).wait()
    pltpu.make_async_copy(buf1, out_hbm.at[pl.ds(0, C)], sem1).wait()


@jax.jit
def kernel(atomic_numbers, positions):
    del positions
    return _onehot_sc(atomic_numbers)


# speedup vs baseline: 1.0087x; 1.0087x over previous
import functools

import jax
import jax.numpy as jnp
from jax import lax
from jax.experimental import pallas as pl
from jax.experimental.pallas import tpu as pltpu
from jax.experimental.pallas import tpu_sc as plsc

N_NODES = 100000
NUM_TYPES = 128

NC = 2
NS = 16
NW = NC * NS

RPW = 3200
LAST_BASE = (NW - 1) * RPW
LAST_ROWS = N_NODES - LAST_BASE
C = 80
GROUPS = C // 16
NB = 3

_mesh = plsc.VectorSubcoreMesh(core_axis_name="c", subcore_axis_name="s")


@functools.partial(
    pl.kernel,
    mesh=_mesh,
    compiler_params=pltpu.CompilerParams(
        needs_layout_passes=False,
        disable_bounds_checks=True,
        disable_semaphore_checks=True,
    ),
    out_type=jax.ShapeDtypeStruct((N_NODES, NUM_TYPES), jnp.float32),
    scratch_types=[
        pltpu.VMEM((RPW,), jnp.int32),
        pltpu.VMEM((NB, C, NUM_TYPES), jnp.float32),
        pltpu.SemaphoreType.DMA,
        pltpu.SemaphoreType.DMA,
        pltpu.SemaphoreType.DMA,
    ],
)
def _onehot_sc(a_hbm, out_hbm, idx_v, bufs, sem0, sem1, sem2):
    wid = lax.axis_index("s") * NC + lax.axis_index("c")
    base = wid * RPW
    rows_w = jnp.minimum(RPW, N_NODES - base)
    nch = rows_w // C
    ntrip = (rows_w + NB * C - 1) // (NB * C)

    @pl.when(wid < NW - 1)
    def _():
        pltpu.make_async_copy(a_hbm.at[pl.ds(base, RPW)], idx_v, sem0).start()

    @pl.when(wid == NW - 1)
    def _():
        pltpu.make_async_copy(a_hbm.at[pl.ds(LAST_BASE, LAST_ROWS)],
                              idx_v.at[pl.ds(0, LAST_ROWS)], sem0).start()

    iota16 = lax.iota(jnp.int32, 16)
    ones16 = jnp.ones((16,), jnp.float32)
    zeros16 = jnp.zeros((16,), jnp.float32)
    sems = (sem0, sem1, sem2)

    def zrow(r, carry):
        for b in range(NB):
            for k in range(NUM_TYPES // 16):
                bufs[b, r, pl.ds(k * 16, 16)] = zeros16
        return carry

    lax.fori_loop(0, C, zrow, 0)

    @pl.when(wid < NW - 1)
    def _():
        pltpu.make_async_copy(a_hbm.at[pl.ds(base, RPW)], idx_v, sem0).wait()

    @pl.when(wid == NW - 1)
    def _():
        pltpu.make_async_copy(a_hbm.at[pl.ds(LAST_BASE, LAST_ROWS)],
                              idx_v.at[pl.ds(0, LAST_ROWS)], sem0).wait()

    def scat(b, ci, vals):
        def g_body(g, carry):
            a = idx_v[pl.ds(ci * C + g * 16, 16)]
            rows = g * 16 + iota16
            plsc.store_scatter(bufs.at[b], [rows, a], vals)
            return carry

        lax.fori_loop(0, GROUPS, g_body, 0)

    def trip(p, carry):
        for b in range(NB):
            ci = NB * p + b
            sem = sems[b]
            row0 = base + ci * C

            @pl.when(ci < nch)
            def _():
                @pl.when(ci >= NB)
                def _():
                    pltpu.make_async_copy(
                        bufs.at[b], out_hbm.at[pl.ds(row0, C)], sem).wait()
                    scat(b, ci - NB, zeros16)

                scat(b, ci, ones16)
                pltpu.make_async_copy(
                    bufs.at[b], out_hbm.at[pl.ds(row0, C)], sem).start()
        return carry

    lax.fori_loop(0, ntrip, trip, 0)

    for b in range(NB):
        pltpu.make_async_copy(bufs.at[b], out_hbm.at[pl.ds(0, C)], sems[b]).wait()


@jax.jit
def kernel(atomic_numbers, positions):
    del positions
    return _onehot_sc(atomic_numbers)


# speedup vs baseline: 1.0233x; 1.0145x over previous
import functools

import jax
import jax.numpy as jnp
from jax import lax
from jax.experimental import pallas as pl
from jax.experimental.pallas import tpu as pltpu
from jax.experimental.pallas import tpu_sc as plsc

N_NODES = 100000
NUM_TYPES = 128

NC = 2
NS = 16
NW = NC * NS

RPW = 3200
LAST_BASE = (NW - 1) * RPW
LAST_ROWS = N_NODES - LAST_BASE
C = 80
GROUPS = C // 16

_mesh = plsc.VectorSubcoreMesh(core_axis_name="c", subcore_axis_name="s")


@functools.partial(
    pl.kernel,
    mesh=_mesh,
    compiler_params=pltpu.CompilerParams(
        needs_layout_passes=False,
        disable_bounds_checks=True,
        disable_semaphore_checks=True,
    ),
    out_type=jax.ShapeDtypeStruct((N_NODES, NUM_TYPES), jnp.float32),
    scratch_types=[
        pltpu.VMEM((RPW,), jnp.int32),
        pltpu.VMEM((C, NUM_TYPES), jnp.float32),
        pltpu.VMEM((C, NUM_TYPES), jnp.float32),
        pltpu.SemaphoreType.DMA,
        pltpu.SemaphoreType.DMA,
    ],
)
def _onehot_sc(a_hbm, out_hbm, idx_v, buf0, buf1, sem0, sem1):
    wid = lax.axis_index("s") * NC + lax.axis_index("c")
    base = wid * RPW
    rows_w = jnp.minimum(RPW, N_NODES - base)
    npairs = rows_w // (2 * C)

    @pl.when(wid < NW - 1)
    def _():
        pltpu.make_async_copy(a_hbm.at[pl.ds(base, RPW)], idx_v, sem0).start()

    @pl.when(wid == NW - 1)
    def _():
        pltpu.make_async_copy(a_hbm.at[pl.ds(LAST_BASE, LAST_ROWS)],
                              idx_v.at[pl.ds(0, LAST_ROWS)], sem0).start()

    iota16 = lax.iota(jnp.int32, 16)
    ones16 = jnp.ones((16,), jnp.float32)
    zeros16 = jnp.zeros((16,), jnp.float32)
    bufs = (buf0, buf1)
    sems = (sem0, sem1)

    def zrow(r, carry):
        for k in range(NUM_TYPES // 16):
            buf0[r, pl.ds(k * 16, 16)] = zeros16
            buf1[r, pl.ds(k * 16, 16)] = zeros16
        return carry

    lax.fori_loop(0, C, zrow, 0)

    @pl.when(wid < NW - 1)
    def _():
        pltpu.make_async_copy(a_hbm.at[pl.ds(base, RPW)], idx_v, sem0).wait()

    @pl.when(wid == NW - 1)
    def _():
        pltpu.make_async_copy(a_hbm.at[pl.ds(LAST_BASE, LAST_ROWS)],
                              idx_v.at[pl.ds(0, LAST_ROWS)], sem0).wait()

    def scat(buf, ci, vals):
        def g_body(g, carry):
            a = idx_v[pl.ds(ci * C + g * 16, 16)]
            rows = g * 16 + iota16
            plsc.store_scatter(buf, [rows, a], vals)
            return carry

        lax.fori_loop(0, GROUPS, g_body, 0)

    def pair(p, carry):
        for b in range(2):
            ci = 2 * p + b
            buf = bufs[b]
            sem = sems[b]
            row0 = base + ci * C

            @pl.when(p > 0)
            def _():
                pltpu.make_async_copy(buf, out_hbm.at[pl.ds(row0, C)], sem).wait()
                scat(buf, ci - 2, zeros16)

            scat(buf, ci, ones16)
            pltpu.make_async_copy(buf, out_hbm.at[pl.ds(row0, C)], sem).start()
        return carry

    lax.fori_loop(0, npairs, pair, 0)

    pltpu.make_async_copy(buf0, out_hbm.at[pl.ds(0, C)], sem0).wait()
    pltpu.make_async_copy(buf1, out_hbm.at[pl.ds(0, C)], sem1).wait()


@jax.jit
def kernel(atomic_numbers, positions):
    del positions
    return _onehot_sc(atomic_numbers)
